# baseline (speedup 1.0000x reference)
import functools

import jax
import jax.numpy as jnp
from jax import lax
from jax.experimental import pallas as pl
from jax.experimental.pallas import tpu as pltpu
from jax.experimental.pallas import tpu_sc as plsc

_G = 64
_NC = 2
_NS = 16
_NW = _NC * _NS
_LANES = 16
_NB = 2000
_NBP = 2048


def _tc_matvec(x, W_rel, W_root, ei1d):
    n, d = x.shape
    steps = n // _NB

    def body(w1_ref, w2_ref, x_ref, ei_ref, o1_ref, o2_ref):
        del ei_ref
        w = jnp.concatenate([w1_ref[...], w2_ref[...]], axis=0)
        y = lax.dot_general(
            w, x_ref[...], (((1,), (1,)), ((), ())),
            precision=lax.Precision.HIGHEST,
            preferred_element_type=jnp.float32)
        y0p = jnp.concatenate(
            [y[0:1, :], jnp.zeros((1, _NBP - _NB), jnp.float32)], axis=1)
        o1_ref[...] = y0p.reshape(1, 1, _NBP)
        o2_ref[...] = y[1:2, :].reshape(1, 1, _NB)

    return pl.pallas_call(
        body,
        grid=(steps,),
        in_specs=[
            pl.BlockSpec((1, d), lambda i: (0, 0)),
            pl.BlockSpec((1, d), lambda i: (0, 0)),
            pl.BlockSpec((_NB, d), lambda i: (i, 0)),
            pl.BlockSpec(memory_space=pl.ANY),
        ],
        out_specs=[
            pl.BlockSpec((1, 1, _NBP), lambda i: (i, 0, 0)),
            pl.BlockSpec((1, 1, _NB), lambda i: (i, 0, 0)),
        ],
        out_shape=[
            jax.ShapeDtypeStruct((steps, 1, _NBP), jnp.float32),
            jax.ShapeDtypeStruct((steps, 1, _NB), jnp.float32),
        ],
    )(W_rel, W_root, x, ei1d)


def _sc_edge_sum(y_rel, ei1d, e, n, ch):
    chv = ch + _LANES - ch % _LANES if ch % _LANES else ch
    nfull = ch // (4 * _LANES)
    rem_v = range(4 * nfull, chv // _LANES)
    nyp = n + _LANES
    seg = -(-n // (_NS * _LANES)) * _LANES
    seg_l = n - seg * (_NS - 1)
    mesh = plsc.VectorSubcoreMesh(
        core_axis_name="c", subcore_axis_name="s",
        num_cores=_NC, num_subcores=_NS)

    @functools.partial(
        pl.kernel,
        out_type=jax.ShapeDtypeStruct((2 * n,), jnp.float32),
        mesh=mesh,
        compiler_params=pltpu.CompilerParams(
            needs_layout_passes=False, use_tc_tiling_on_sc=False),
        scratch_types=[
            pltpu.VMEM((nyp,), jnp.float32),
            pltpu.VMEM((n,), jnp.float32),
            pltpu.VMEM((chv,), jnp.int32),
            pltpu.VMEM((chv,), jnp.int32),
            pltpu.VMEM((_NS, seg), jnp.float32),
            pltpu.VMEM((seg,), jnp.float32),
            pltpu.VMEM_SHARED((_NS, n), jnp.float32),
            pltpu.SemaphoreType.DMA,
        ],
    )
    def k(y_hbm, ei_hbm, out_hbm,
          y_v, acc_v, src_v, dst_v, tmp_v, red_v, shared, sem):
        cid = lax.axis_index("c")
        sid = lax.axis_index("s")
        wid = cid * _NS + sid
        base = pl.multiple_of(wid * ch, 8)
        if chv != ch:
            src_v[pl.ds(chv - _LANES, _LANES)] = jnp.full(
                (_LANES,), n, jnp.int32)
            dst_v[pl.ds(chv - _LANES, _LANES)] = jnp.zeros(
                (_LANES,), jnp.int32)
        copies = [
            pltpu.async_copy(y_hbm.at[pl.ds(kk * _NBP, _NB)],
                             y_v.at[pl.ds(kk * _NB, _NB)], sem)
            for kk in range(n // _NB)
        ]
        y_v[pl.ds(n, _LANES)] = jnp.zeros((_LANES,), jnp.float32)
        for cp in copies:
            cp.wait()
        pltpu.sync_copy(ei_hbm.at[pl.ds(base, ch)], src_v.at[pl.ds(0, ch)])
        pltpu.sync_copy(ei_hbm.at[pl.ds(e + base, ch)],
                        dst_v.at[pl.ds(0, ch)])

        @plsc.parallel_loop(0, n // _LANES, unroll=8)
        def zero_body(i):
            acc_v[pl.ds(i * _LANES, _LANES)] = jnp.zeros(
                (_LANES,), jnp.float32)

        @plsc.parallel_loop(0, chv // _LANES, unroll=32)
        def edge_body(i):
            off = i * _LANES
            s = src_v[pl.ds(off, _LANES)]
            dd = dst_v[pl.ds(off, _LANES)]
            v = plsc.load_gather(y_v, [s])
            plsc.addupdate_scatter(acc_v, [dd], v)

        pltpu.sync_copy(acc_v, shared.at[sid])
        plsc.subcore_barrier()

        def reduce_slice(length):
            off = pl.multiple_of(sid * seg, 8)
            pltpu.sync_copy(shared.at[:, pl.ds(off, length)],
                            tmp_v.at[:, pl.ds(0, length)])

            @plsc.parallel_loop(0, length // _LANES, unroll=2)
            def red_body(v):
                accv = tmp_v[0, pl.ds(v * _LANES, _LANES)]
                for p in range(1, _NS):
                    accv = accv + tmp_v[p, pl.ds(v * _LANES, _LANES)]
                red_v[pl.ds(v * _LANES, _LANES)] = accv
            pltpu.sync_copy(
                red_v.at[pl.ds(0, length)],
                out_hbm.at[pl.ds(cid * n + off, length)])

        @pl.when(sid < _NS - 1)
        def _():
            reduce_slice(seg)

        @pl.when(sid == _NS - 1)
        def _():
            reduce_slice(seg_l)

    return k(y_rel, ei1d)


def _tc_softmax_pool(x, xc2, y_root3, batch3, bias2d):
    n, d = x.shape
    steps = n // _NB

    def body(xcp_ref, yr_ref, bat_ref, b_ref, x_ref, o_ref, xc_ref, sc_ref):
        j = pl.program_id(0)
        neg = jnp.float32(-1e30)

        @pl.when(j == 0)
        def _():
            seg_max = jnp.full((_G, 1), neg, jnp.float32)
            a = xcp_ref[...]
            conv10 = a[0:n] + a[n:2 * n]
            for jb in range(steps):
                conv = conv10[jb * _NB:(jb + 1) * _NB].reshape(1, _NB)
                xc = conv + yr_ref[jb] + b_ref[0, 0]
                xc_ref[jb] = xc
                m = (bat_ref[jb] == lax.broadcasted_iota(
                    jnp.int32, (_G, _NB), 0))
                seg_max = jnp.maximum(
                    seg_max,
                    jnp.max(jnp.where(m, xc, neg), axis=1, keepdims=True))
            denom = jnp.zeros((_G, 1), jnp.float32)
            for jb in range(steps):
                m = (bat_ref[jb] == lax.broadcasted_iota(
                    jnp.int32, (_G, _NB), 0)).astype(jnp.float32)
                shift = xc_ref[jb] - jnp.sum(m * seg_max, axis=0,
                                             keepdims=True)
                ex = jnp.exp(jnp.minimum(shift, 0.0))
                xc_ref[jb] = ex
                denom = denom + jnp.sum(m * ex, axis=1, keepdims=True)
            inv = 1.0 / (denom + 1e-16)
            for jb in range(steps):
                m = (bat_ref[jb] == lax.broadcasted_iota(
                    jnp.int32, (_G, _NB), 0)).astype(jnp.float32)
                sc_ref[jb] = xc_ref[jb] * jnp.sum(m * inv, axis=0,
                                                  keepdims=True)
            o_ref[...] = jnp.zeros_like(o_ref)

        m = (bat_ref[j] == lax.broadcasted_iota(
            jnp.int32, (_G, _NB), 0)).astype(jnp.float32)
        p = m * sc_ref[j]
        o_ref[...] += lax.dot_general(
            p, x_ref[...], (((1,), (0,)), ((), ())),
            preferred_element_type=jnp.float32)

    return pl.pallas_call(
        body,
        grid=(steps,),
        in_specs=[
            pl.BlockSpec((2 * n,), lambda i: (0,)),
            pl.BlockSpec((steps, 1, _NB), lambda i: (0, 0, 0)),
            pl.BlockSpec((steps, 1, _NB), lambda i: (0, 0, 0)),
            pl.BlockSpec((1, 1), lambda i: (0, 0)),
            pl.BlockSpec((_NB, d), lambda i: (i, 0)),
        ],
        out_specs=pl.BlockSpec((_G, d), lambda i: (0, 0)),
        out_shape=jax.ShapeDtypeStruct((_G, d), jnp.float32),
        scratch_shapes=[
            pltpu.VMEM((steps, 1, _NB), jnp.float32),
            pltpu.VMEM((steps, 1, _NB), jnp.float32),
        ],
    )(xc2, y_root3, batch3, bias2d, x)


def kernel(x, edge_index, batch, W_rel, b_rel, W_root):
    n, d = x.shape
    e = edge_index.shape[1]
    steps = n // _NB
    ch = -(-e // (_NW * 8)) * 8

    batch3 = batch.reshape(steps, 1, _NB)
    bias2d = b_rel.reshape(1, 1)
    ei1d = edge_index.reshape(2 * e)

    y_rel3, y_root3 = _tc_matvec(x, W_rel, W_root, ei1d)
    xc2 = _sc_edge_sum(y_rel3.reshape(steps * _NBP), ei1d, e, n, ch)
    gx = _tc_softmax_pool(x, xc2, y_root3, batch3, bias2d)
    return gx


# speedup vs baseline: 1.0040x; 1.0040x over previous
import functools

import jax
import jax.numpy as jnp
from jax import lax
from jax.experimental import pallas as pl
from jax.experimental.pallas import tpu as pltpu
from jax.experimental.pallas import tpu_sc as plsc

_G = 64
_NC = 2
_NS = 16
_NW = _NC * _NS
_LANES = 16
_NB = 2000
_NBP = 2048


def _tc_matvec(x, W_rel, W_root, ei1d):
    n, d = x.shape
    steps = n // _NB

    def body(w1_ref, w2_ref, x_ref, ei_ref, o1_ref, o2_ref):
        del ei_ref
        w = jnp.concatenate([w1_ref[...], w2_ref[...]], axis=0)
        y = lax.dot_general(
            w, x_ref[...], (((1,), (1,)), ((), ())),
            precision=lax.Precision.HIGHEST,
            preferred_element_type=jnp.float32)
        y0p = jnp.concatenate(
            [y[0:1, :], jnp.zeros((1, _NBP - _NB), jnp.float32)], axis=1)
        o1_ref[...] = y0p.reshape(1, 1, _NBP)
        o2_ref[...] = y[1:2, :].reshape(1, 1, _NB)

    return pl.pallas_call(
        body,
        grid=(steps,),
        in_specs=[
            pl.BlockSpec((1, d), lambda i: (0, 0)),
            pl.BlockSpec((1, d), lambda i: (0, 0)),
            pl.BlockSpec((_NB, d), lambda i: (i, 0)),
            pl.BlockSpec(memory_space=pl.ANY),
        ],
        out_specs=[
            pl.BlockSpec((1, 1, _NBP), lambda i: (i, 0, 0)),
            pl.BlockSpec((1, 1, _NB), lambda i: (i, 0, 0)),
        ],
        out_shape=[
            jax.ShapeDtypeStruct((steps, 1, _NBP), jnp.float32),
            jax.ShapeDtypeStruct((steps, 1, _NB), jnp.float32),
        ],
    )(W_rel, W_root, x, ei1d)


def _sc_edge_sum(y_rel, ei1d, e, n, ch):
    chv = ch + _LANES - ch % _LANES if ch % _LANES else ch
    nfull = ch // (4 * _LANES)
    rem_v = range(4 * nfull, chv // _LANES)
    nyp = n + _LANES
    seg = -(-n // (_NS * _LANES)) * _LANES
    seg_l = n - seg * (_NS - 1)
    mesh = plsc.VectorSubcoreMesh(
        core_axis_name="c", subcore_axis_name="s",
        num_cores=_NC, num_subcores=_NS)

    @functools.partial(
        pl.kernel,
        out_type=jax.ShapeDtypeStruct((2 * n,), jnp.float32),
        mesh=mesh,
        compiler_params=pltpu.CompilerParams(
            needs_layout_passes=False, use_tc_tiling_on_sc=False),
        scratch_types=[
            pltpu.VMEM((nyp,), jnp.float32),
            pltpu.VMEM((n,), jnp.float32),
            pltpu.VMEM((chv,), jnp.int32),
            pltpu.VMEM((chv,), jnp.int32),
            pltpu.VMEM((_NS, seg), jnp.float32),
            pltpu.VMEM((seg,), jnp.float32),
            pltpu.VMEM_SHARED((_NS, n), jnp.float32),
            pltpu.SemaphoreType.DMA,
        ],
    )
    def k(y_hbm, ei_hbm, out_hbm,
          y_v, acc_v, src_v, dst_v, tmp_v, red_v, shared, sem):
        cid = lax.axis_index("c")
        sid = lax.axis_index("s")
        wid = cid * _NS + sid
        base = pl.multiple_of(wid * ch, 8)
        if chv != ch:
            src_v[pl.ds(chv - _LANES, _LANES)] = jnp.full(
                (_LANES,), n, jnp.int32)
            dst_v[pl.ds(chv - _LANES, _LANES)] = jnp.zeros(
                (_LANES,), jnp.int32)
        copies = [
            pltpu.async_copy(y_hbm.at[pl.ds(kk * _NBP, _NB)],
                             y_v.at[pl.ds(kk * _NB, _NB)], sem)
            for kk in range(n // _NB)
        ]
        y_v[pl.ds(n, _LANES)] = jnp.zeros((_LANES,), jnp.float32)
        for cp in copies:
            cp.wait()
        pltpu.sync_copy(ei_hbm.at[pl.ds(base, ch)], src_v.at[pl.ds(0, ch)])
        pltpu.sync_copy(ei_hbm.at[pl.ds(e + base, ch)],
                        dst_v.at[pl.ds(0, ch)])

        @plsc.parallel_loop(0, n // _LANES, unroll=8)
        def zero_body(i):
            acc_v[pl.ds(i * _LANES, _LANES)] = jnp.zeros(
                (_LANES,), jnp.float32)

        @plsc.parallel_loop(0, chv // _LANES, unroll=16)
        def edge_body(i):
            off = i * _LANES
            s = src_v[pl.ds(off, _LANES)]
            dd = dst_v[pl.ds(off, _LANES)]
            v = plsc.load_gather(y_v, [s])
            plsc.addupdate_scatter(acc_v, [dd], v)

        pltpu.sync_copy(acc_v, shared.at[sid])
        plsc.subcore_barrier()

        def reduce_slice(length):
            off = pl.multiple_of(sid * seg, 8)
            pltpu.sync_copy(shared.at[:, pl.ds(off, length)],
                            tmp_v.at[:, pl.ds(0, length)])

            @plsc.parallel_loop(0, length // _LANES, unroll=2)
            def red_body(v):
                accv = tmp_v[0, pl.ds(v * _LANES, _LANES)]
                for p in range(1, _NS):
                    accv = accv + tmp_v[p, pl.ds(v * _LANES, _LANES)]
                red_v[pl.ds(v * _LANES, _LANES)] = accv
            pltpu.sync_copy(
                red_v.at[pl.ds(0, length)],
                out_hbm.at[pl.ds(cid * n + off, length)])

        @pl.when(sid < _NS - 1)
        def _():
            reduce_slice(seg)

        @pl.when(sid == _NS - 1)
        def _():
            reduce_slice(seg_l)

    return k(y_rel, ei1d)


def _tc_softmax_pool(x, xc2, y_root3, batch3, bias2d):
    n, d = x.shape
    steps = n // _NB

    def body(xcp_ref, yr_ref, bat_ref, b_ref, x_ref, o_ref, xc_ref, sc_ref):
        j = pl.program_id(0)
        neg = jnp.float32(-1e30)

        @pl.when(j == 0)
        def _():
            seg_max = jnp.full((_G, 1), neg, jnp.float32)
            a = xcp_ref[...]
            conv10 = a[0:n] + a[n:2 * n]
            for jb in range(steps):
                conv = conv10[jb * _NB:(jb + 1) * _NB].reshape(1, _NB)
                xc = conv + yr_ref[jb] + b_ref[0, 0]
                xc_ref[jb] = xc
                m = (bat_ref[jb] == lax.broadcasted_iota(
                    jnp.int32, (_G, _NB), 0))
                seg_max = jnp.maximum(
                    seg_max,
                    jnp.max(jnp.where(m, xc, neg), axis=1, keepdims=True))
            denom = jnp.zeros((_G, 1), jnp.float32)
            for jb in range(steps):
                m = (bat_ref[jb] == lax.broadcasted_iota(
                    jnp.int32, (_G, _NB), 0)).astype(jnp.float32)
                shift = xc_ref[jb] - jnp.sum(m * seg_max, axis=0,
                                             keepdims=True)
                ex = jnp.exp(jnp.minimum(shift, 0.0))
                xc_ref[jb] = ex
                denom = denom + jnp.sum(m * ex, axis=1, keepdims=True)
            inv = 1.0 / (denom + 1e-16)
            for jb in range(steps):
                m = (bat_ref[jb] == lax.broadcasted_iota(
                    jnp.int32, (_G, _NB), 0)).astype(jnp.float32)
                sc_ref[jb] = xc_ref[jb] * jnp.sum(m * inv, axis=0,
                                                  keepdims=True)
            o_ref[...] = jnp.zeros_like(o_ref)

        m = (bat_ref[j] == lax.broadcasted_iota(
            jnp.int32, (_G, _NB), 0)).astype(jnp.float32)
        p = m * sc_ref[j]
        o_ref[...] += lax.dot_general(
            p, x_ref[...], (((1,), (0,)), ((), ())),
            preferred_element_type=jnp.float32)

    return pl.pallas_call(
        body,
        grid=(steps,),
        in_specs=[
            pl.BlockSpec((2 * n,), lambda i: (0,)),
            pl.BlockSpec((steps, 1, _NB), lambda i: (0, 0, 0)),
            pl.BlockSpec((steps, 1, _NB), lambda i: (0, 0, 0)),
            pl.BlockSpec((1, 1), lambda i: (0, 0)),
            pl.BlockSpec((_NB, d), lambda i: (i, 0)),
        ],
        out_specs=pl.BlockSpec((_G, d), lambda i: (0, 0)),
        out_shape=jax.ShapeDtypeStruct((_G, d), jnp.float32),
        scratch_shapes=[
            pltpu.VMEM((steps, 1, _NB), jnp.float32),
            pltpu.VMEM((steps, 1, _NB), jnp.float32),
        ],
    )(xc2, y_root3, batch3, bias2d, x)


def kernel(x, edge_index, batch, W_rel, b_rel, W_root):
    n, d = x.shape
    e = edge_index.shape[1]
    steps = n // _NB
    ch = -(-e // (_NW * 8)) * 8

    batch3 = batch.reshape(steps, 1, _NB)
    bias2d = b_rel.reshape(1, 1)
    ei1d = edge_index.reshape(2 * e)

    y_rel3, y_root3 = _tc_matvec(x, W_rel, W_root, ei1d)
    xc2 = _sc_edge_sum(y_rel3.reshape(steps * _NBP), ei1d, e, n, ch)
    gx = _tc_softmax_pool(x, xc2, y_root3, batch3, bias2d)
    return gx
